# Initial kernel scaffold; baseline (speedup 1.0000x reference)
import jax
import jax.numpy as jnp
from jax.experimental import pallas as pl


def kernel(x, edge_index, batch, W0, b0, W1, b1, W2, b2, L1_W, L1_b, L2_W, L2_b):
    raise NotImplementedError("write your pallas kernel here")


# speedup vs baseline: 7.1227x; 7.1227x over previous
import jax
import jax.numpy as jnp
from jax import lax
from jax.experimental import pallas as pl
from jax.experimental.pallas import tpu as pltpu
from jax.experimental.pallas import tpu_sc as plsc

_NCORES = 2
_NSUB = 16
_NW = _NCORES * _NSUB
_WIN = 128
_HW = 64
_GRAPHS = 16


def _sc_mesh():
    return plsc.VectorSubcoreMesh(core_axis_name="c", subcore_axis_name="s")


_CHUNK = 8


def _make_deg_kernel(n_pad, windows):
    rps = n_pad // _NSUB
    nchunks = windows // _CHUNK

    def body(dst_hbm, ones_hbm, zeros_hbm, out_hbm, dstv, onesv, acc_sh):
        cid = lax.axis_index("c")
        sid = lax.axis_index("s")
        wid = sid * _NCORES + cid
        pltpu.sync_copy(zeros_hbm.at[pl.ds(sid * rps, rps)],
                        acc_sh.at[pl.ds(sid * rps, rps)])
        pltpu.sync_copy(ones_hbm, onesv)
        plsc.subcore_barrier()

        @pl.loop(0, nchunks)
        def _(c):
            pltpu.sync_copy(dst_hbm.at[wid, pl.ds(c * _CHUNK, _CHUNK)], dstv)
            for j in range(_CHUNK):
                pltpu.sync_copy(onesv, acc_sh.at[dstv.at[j]], add=True)

        plsc.subcore_barrier()
        pltpu.sync_copy(acc_sh.at[pl.ds(sid * rps, rps)],
                        out_hbm.at[cid, pl.ds(sid * rps, rps)])

    return pl.kernel(
        body,
        out_type=jax.ShapeDtypeStruct((_NCORES, n_pad, 128), jnp.float32),
        mesh=_sc_mesh(),
        scratch_types=[
            pltpu.VMEM((_CHUNK, _WIN), jnp.int32),
            pltpu.VMEM((_WIN, 128), jnp.float32),
            pltpu.VMEM_SHARED((n_pad, 128), jnp.float32),
        ],
    )


def _make_edge_kernel(n_pad, windows, d):
    rps = n_pad // _NSUB
    nchunks = windows // _CHUNK

    def body(g_hbm, src_hbm, dst_hbm, zeros_hbm, out_hbm,
             srcv, dstv, rows_a, rows_b, acc_sh, sem_a, sem_b):
        cid = lax.axis_index("c")
        sid = lax.axis_index("s")
        wid = sid * _NCORES + cid
        pltpu.sync_copy(zeros_hbm.at[pl.ds(sid * rps, rps)],
                        acc_sh.at[pl.ds(sid * rps, rps)])
        plsc.subcore_barrier()

        rows = (rows_a, rows_b)
        sems = (sem_a, sem_b)

        @pl.loop(0, nchunks)
        def _(c):
            pltpu.sync_copy(src_hbm.at[wid, pl.ds(c * _CHUNK, _CHUNK)], srcv)
            pltpu.sync_copy(dst_hbm.at[wid, pl.ds(c * _CHUNK, _CHUNK)], dstv)
            pltpu.async_copy(g_hbm.at[srcv.at[0]], rows[0], sems[0])
            for j in range(_CHUNK):
                if j + 1 < _CHUNK:
                    pltpu.async_copy(g_hbm.at[srcv.at[j + 1]],
                                     rows[(j + 1) % 2], sems[(j + 1) % 2])
                pltpu.make_async_copy(g_hbm.at[srcv.at[j]],
                                      rows[j % 2], sems[j % 2]).wait()
                pltpu.sync_copy(rows[j % 2], acc_sh.at[dstv.at[j]], add=True)

        plsc.subcore_barrier()
        pltpu.sync_copy(acc_sh.at[pl.ds(sid * rps, rps)],
                        out_hbm.at[cid, pl.ds(sid * rps, rps)])

    return pl.kernel(
        body,
        out_type=jax.ShapeDtypeStruct((_NCORES, n_pad, d), jnp.float32),
        mesh=_sc_mesh(),
        scratch_types=[
            pltpu.VMEM((_CHUNK, _WIN), jnp.int32),
            pltpu.VMEM((_CHUNK, _WIN), jnp.int32),
            pltpu.VMEM((_WIN, d), jnp.float32),
            pltpu.VMEM((_WIN, d), jnp.float32),
            pltpu.VMEM_SHARED((n_pad, d), jnp.float32),
            pltpu.SemaphoreType.DMA,
            pltpu.SemaphoreType.DMA,
        ],
    )


def _dense0_body(x_ref, w_ref, o_ref):
    o_ref[...] = jnp.dot(x_ref[...], w_ref[...],
                         preferred_element_type=jnp.float32)


def _dense0(xp, w):
    return pl.pallas_call(
        _dense0_body,
        out_shape=jax.ShapeDtypeStruct(xp.shape, jnp.float32),
    )(xp, w)


def _prep(deg_parts, hw, n):
    n_pad, d = hw.shape

    def body(dp_ref, hw_ref, dinv_ref, g_ref):
        rows = lax.broadcasted_iota(jnp.int32, (n_pad, 1), 0)
        real = rows < n
        dp = dp_ref[...]
        deg = dp[0, :, :1] + dp[1, :, :1] + jnp.where(real, 1.0, 0.0)
        dinv = jnp.where(real, lax.rsqrt(jnp.maximum(deg, 1e-12)), 0.0)
        dinv_ref[...] = dinv
        g_ref[...] = hw_ref[...] * dinv

    return pl.pallas_call(
        body,
        out_shape=[jax.ShapeDtypeStruct((n_pad, 1), jnp.float32),
                   jax.ShapeDtypeStruct((n_pad, d), jnp.float32)],
    )(deg_parts, hw)


def _combine(p0, p1, g, dinv, b, w_next):
    n_pad, d = g.shape

    def body(p0_ref, p1_ref, g_ref, dinv_ref, b_ref, w_ref, o_ref):
        acc = p0_ref[...] + p1_ref[...] + g_ref[...]
        dinv = dinv_ref[...]
        h = jnp.maximum(dinv * acc + b_ref[...], 0.0)
        o_ref[...] = dinv * jnp.dot(h, w_ref[...],
                                    preferred_element_type=jnp.float32)

    return pl.pallas_call(
        body,
        out_shape=jax.ShapeDtypeStruct((n_pad, d), jnp.float32),
    )(p0, p1, g, dinv, b, w_next)


def _final(p0, p1, g, dinv, b, batchp, l1w, l1b, l2w, l2b):
    n_pad, d = g.shape
    out_d = l2w.shape[1]

    def body(p0_ref, p1_ref, g_ref, dinv_ref, b_ref, bat_ref,
             l1w_ref, l1b_ref, l2w_ref, l2b_ref, o_ref):
        acc = p0_ref[...] + p1_ref[...] + g_ref[...]
        h = jnp.maximum(dinv_ref[...] * acc + b_ref[...], 0.0)
        gid = lax.broadcasted_iota(jnp.int32, (_GRAPHS, n_pad), 0)
        oh = (bat_ref[...] == gid).astype(jnp.float32)
        sums = jnp.dot(oh, h, preferred_element_type=jnp.float32)
        cnt = jnp.sum(oh, axis=1, keepdims=True)
        pooled = sums / jnp.maximum(cnt, 1.0)
        a1 = jnp.maximum(jnp.dot(pooled, l1w_ref[...],
                                 preferred_element_type=jnp.float32)
                         + l1b_ref[...], 0.0)
        o_ref[...] = jnp.dot(a1, l2w_ref[...],
                             preferred_element_type=jnp.float32) + l2b_ref[...]

    return pl.pallas_call(
        body,
        out_shape=jax.ShapeDtypeStruct((_GRAPHS, out_d), jnp.float32),
    )(p0, p1, g, dinv, b, batchp, l1w, l1b, l2w, l2b)


def kernel(x, edge_index, batch, W0, b0, W1, b1, W2, b2, L1_W, L1_b, L2_W, L2_b):
    n, d = x.shape
    e = edge_index.shape[1]
    n_pad = -(-(n + 1) // 2048) * 2048

    xp = jnp.pad(x, ((0, n_pad - n), (0, 0)))
    block = _NW * _WIN
    w = -(-e // block)
    w = -(-w // _CHUNK) * _CHUNK
    fill = jnp.full((w * block - e,), n, jnp.int32)
    srcp = jnp.concatenate([edge_index[0], fill]).reshape(_NW, w, _WIN)
    dstp = jnp.concatenate([edge_index[1], fill]).reshape(_NW, w, _WIN)
    batchp = jnp.concatenate(
        [batch, jnp.full((n_pad - n,), _GRAPHS, jnp.int32)]).reshape(1, n_pad)

    ones_deg = jnp.ones((_WIN, 128), jnp.float32)
    zeros_rows = jnp.zeros((n_pad, d), jnp.float32)
    b0r, b1r, b2r = b0.reshape(1, -1), b1.reshape(1, -1), b2.reshape(1, -1)
    l1br, l2br = L1_b.reshape(1, -1), L2_b.reshape(1, -1)

    deg_parts = _make_deg_kernel(n_pad, w)(dstp, ones_deg, zeros_rows)
    hw0 = _dense0(xp, W0)
    dinv, g = _prep(deg_parts, hw0, n)

    edge_k = _make_edge_kernel(n_pad, w, d)
    p = edge_k(g, srcp, dstp, zeros_rows)
    g = _combine(p[0], p[1], g, dinv, b0r, W1)
    p = edge_k(g, srcp, dstp, zeros_rows)
    g = _combine(p[0], p[1], g, dinv, b1r, W2)
    p = edge_k(g, srcp, dstp, zeros_rows)
    return _final(p[0], p[1], g, dinv, b2r, batchp, L1_W, l1br, L2_W, l2br)
